# baseline (speedup 1.0000x reference)
import functools

import jax
import jax.numpy as jnp
from jax.experimental import pallas as pl

F = 128
LEAK = 0.1


def _dot(a, b):
    return jnp.dot(a, b, preferred_element_type=jnp.float32)


def _dotm(a, x):
    return jax.lax.dot_general(a, x, (((1,), (0,)), ((), ())),
                               preferred_element_type=jnp.float32)


def _bf(x):
    return x.astype(jnp.bfloat16)


def _leaky(x):
    return jnp.where(x >= 0, x, LEAK * x)


def _block_rows(n, bm):
    return bm if n % bm == 0 else n


def _pre_body(su, ss, sv, tu, ts, tv, w1s, w2s, w3s, w1t, w2t, w3t,
              a_s, b_s, a_t, b_t):
    a_s[:, :F] = _bf(_dot(su[...], w1s[...]))
    a_s[:, F:] = _bf(_dot(ss[...], w2s[...]))
    b_s[...] = _bf(_dot(sv[...], w3s[...]))
    a_t[:, :F] = _bf(_dot(tu[...], w1t[...]))
    a_t[:, F:] = _bf(_dot(ts[...], w2t[...]))
    b_t[...] = _bf(_dot(tv[...], w3t[...]))


def _p1_half(vu, a_in, bias, wg3m, wsg2, y_out):
    h = _leaky(_dotm(vu[...], a_in[...]) + bias[...])
    y_out[:, :F] = _bf(_dot(h[:, :F], wg3m[...]))
    y_out[:, F:] = _bf(_dot(h[:, F:], wsg2[...]))


def _p1_body(vu_s, vu_t, a_s, a_t, bias_s, bias_t,
             w3m_s, wsg2_s, w3m_t, wsg2_t, y_s, y_t):
    _p1_half(vu_s, a_s, bias_s, w3m_s, wsg2_s, y_s)
    _p1_half(vu_t, a_t, bias_t, w3m_t, wsg2_t, y_t)


def _p2_half(uv, b_in, y_in, bias_b, bias_y, ufea, wuum, buum, wsum, bsum,
             wg4m, user_out, d_out):
    blk = uv[...]
    item_ho = _leaky(_dotm(blk, b_in[...]) + bias_b[...])
    h = _leaky(_dotm(blk, y_in[...]) + bias_y[...])
    u_mean = h[:, :F]
    sh2 = h[:, F:]
    slu = _dot(u_mean, wuum[:F, :]) + _dot(ufea[...], wuum[F:, :]) + buum[...]
    csm = _dot(sh2, wsum[:F, :]) + _dot(slu, wsum[F:, :]) + bsum[...]
    user_out[...] = csm + slu
    d_out[...] = _bf(_dot(item_ho, wg4m[...]))


def _p2_body(uv_s, uv_t, b_s, y_s, b_t, y_t,
             bb_s, by_s, bb_t, by_t, ufea_s, ufea_t,
             wuum_s, buum_s, wsum_s, bsum_s, wg4m_s,
             wuum_t, buum_t, wsum_t, bsum_t, wg4m_t,
             user_s, d_s, user_t, d_t):
    _p2_half(uv_s, b_s, y_s, bb_s, by_s, ufea_s,
             wuum_s, buum_s, wsum_s, bsum_s, wg4m_s, user_s, d_s)
    _p2_half(uv_t, b_t, y_t, bb_t, by_t, ufea_t,
             wuum_t, buum_t, wsum_t, bsum_t, wg4m_t, user_t, d_t)


def _p3_half(vu, d_in, bias, vfea, wium, bium, item_out):
    h = _leaky(_dotm(vu[...], d_in[...]) + bias[...])
    item_out[...] = _dot(h, wium[:F, :]) + _dot(vfea[...], wium[F:, :]) + bium[...]


def _p3_body(vu_s, vu_t, d_s, d_t, bias_s, bias_t, vfea_s, vfea_t,
             wium_s, bium_s, wium_t, bium_t, item_s, item_t):
    _p3_half(vu_s, d_s, bias_s, vfea_s, wium_s, bium_s, item_s)
    _p3_half(vu_t, d_t, bias_t, vfea_t, wium_t, bium_t, item_t)


def _const_spec(shape):
    return pl.BlockSpec(shape, lambda i: (0,) * len(shape))


def _row_spec(bm, width):
    return pl.BlockSpec((bm, width), lambda i: (i, 0))


def _row1(b):
    return b.reshape(1, -1)


@functools.partial(jax.jit)
def kernel(source_UV, source_VU, target_UV, target_VU, params):
    cond = params["cond"]
    f32 = jnp.float32
    bf16 = jnp.bfloat16
    s_spec, t_spec = params["src_specific"], params["tgt_specific"]
    su, sv = params["src_user_emb"], params["src_item_emb"]
    tu, tv = params["tgt_user_emb"], params["tgt_item_emb"]
    nu = su.shape[0]
    ni = sv.shape[0]

    bm = _block_rows(nu, 1000)
    a_s, b_s, a_t, b_t = pl.pallas_call(
        _pre_body,
        grid=(nu // bm,),
        in_specs=[_row_spec(bm, F)] * 6 + [_const_spec((F, F))] * 6,
        out_specs=[_row_spec(bm, 2 * F), _row_spec(bm, F),
                   _row_spec(bm, 2 * F), _row_spec(bm, F)],
        out_shape=[jax.ShapeDtypeStruct((nu, 2 * F), bf16),
                   jax.ShapeDtypeStruct((ni, F), bf16),
                   jax.ShapeDtypeStruct((nu, 2 * F), bf16),
                   jax.ShapeDtypeStruct((ni, F), bf16)],
    )(su, params["src_user_share"], sv, tu, params["tgt_user_share"], tv,
      s_spec["gc1"]["W"], cond["s_gc1"]["W"], s_spec["gc2"]["W"],
      t_spec["gc1"]["W"], cond["t_gc1"]["W"], t_spec["gc2"]["W"])

    bm = _block_rows(ni, 200)
    bias1_s = jnp.concatenate([_row1(s_spec["gc1"]["b"]),
                               _row1(cond["s_gc1"]["b"])], axis=1)
    bias1_t = jnp.concatenate([_row1(t_spec["gc1"]["b"]),
                               _row1(cond["t_gc1"]["b"])], axis=1)
    y_s, y_t = pl.pallas_call(
        _p1_body,
        grid=(ni // bm,),
        in_specs=[_row_spec(bm, nu), _row_spec(bm, nu),
                  _const_spec((nu, 2 * F)), _const_spec((nu, 2 * F)),
                  _const_spec((1, 2 * F)), _const_spec((1, 2 * F)),
                  _const_spec((F, F)), _const_spec((F, F)),
                  _const_spec((F, F)), _const_spec((F, F))],
        out_specs=[_row_spec(bm, 2 * F), _row_spec(bm, 2 * F)],
        out_shape=[jax.ShapeDtypeStruct((ni, 2 * F), bf16),
                   jax.ShapeDtypeStruct((ni, 2 * F), bf16)],
    )(source_VU, target_VU, a_s, a_t, bias1_s, bias1_t,
      s_spec["gc3m"]["W"], cond["s_gc2"]["W"],
      t_spec["gc3m"]["W"], cond["t_gc2"]["W"])

    bm = _block_rows(nu, 200)
    by_s = jnp.concatenate([_row1(s_spec["gc3m"]["b"]),
                            _row1(cond["s_gc2"]["b"])], axis=1)
    by_t = jnp.concatenate([_row1(t_spec["gc3m"]["b"]),
                            _row1(cond["t_gc2"]["b"])], axis=1)
    user_s, d_s, user_t, d_t = pl.pallas_call(
        _p2_body,
        grid=(nu // bm,),
        in_specs=[_row_spec(bm, ni), _row_spec(bm, ni),
                  _const_spec((ni, F)), _const_spec((ni, 2 * F)),
                  _const_spec((ni, F)), _const_spec((ni, 2 * F)),
                  _const_spec((1, F)), _const_spec((1, 2 * F)),
                  _const_spec((1, F)), _const_spec((1, 2 * F)),
                  _row_spec(bm, F), _row_spec(bm, F),
                  _const_spec((2 * F, F)), _const_spec((1, F)),
                  _const_spec((2 * F, F)), _const_spec((1, F)),
                  _const_spec((F, F)),
                  _const_spec((2 * F, F)), _const_spec((1, F)),
                  _const_spec((2 * F, F)), _const_spec((1, F)),
                  _const_spec((F, F))],
        out_specs=[_row_spec(bm, F), _row_spec(bm, F),
                   _row_spec(bm, F), _row_spec(bm, F)],
        out_shape=[jax.ShapeDtypeStruct((nu, F), f32),
                   jax.ShapeDtypeStruct((nu, F), bf16),
                   jax.ShapeDtypeStruct((nu, F), f32),
                   jax.ShapeDtypeStruct((nu, F), bf16)],
    )(source_UV, target_UV, b_s, y_s, b_t, y_t,
      _row1(s_spec["gc2"]["b"]), by_s, _row1(t_spec["gc2"]["b"]), by_t,
      su, tu,
      s_spec["uum"]["W"], _row1(s_spec["uum"]["b"]),
      cond["s_um"]["W"], _row1(cond["s_um"]["b"]), s_spec["gc4m"]["W"],
      t_spec["uum"]["W"], _row1(t_spec["uum"]["b"]),
      cond["t_um"]["W"], _row1(cond["t_um"]["b"]), t_spec["gc4m"]["W"])

    bm = _block_rows(ni, 200)
    item_s, item_t = pl.pallas_call(
        _p3_body,
        grid=(ni // bm,),
        in_specs=[_row_spec(bm, nu), _row_spec(bm, nu),
                  _const_spec((nu, F)), _const_spec((nu, F)),
                  _const_spec((1, F)), _const_spec((1, F)),
                  _row_spec(bm, F), _row_spec(bm, F),
                  _const_spec((2 * F, F)), _const_spec((1, F)),
                  _const_spec((2 * F, F)), _const_spec((1, F))],
        out_specs=[_row_spec(bm, F), _row_spec(bm, F)],
        out_shape=[jax.ShapeDtypeStruct((ni, F), f32),
                   jax.ShapeDtypeStruct((ni, F), f32)],
    )(source_VU, target_VU, d_s, d_t,
      _row1(s_spec["gc4m"]["b"]), _row1(t_spec["gc4m"]["b"]), sv, tv,
      s_spec["ium"]["W"], _row1(s_spec["ium"]["b"]),
      t_spec["ium"]["W"], _row1(t_spec["ium"]["b"]))

    return user_s, item_s, user_t, item_t


# speedup vs baseline: 1.0629x; 1.0629x over previous
import functools

import jax
import jax.numpy as jnp
from jax.experimental import pallas as pl

F = 128
LEAK = 0.1


def _dot(a, b):
    return jnp.dot(a, b, preferred_element_type=jnp.float32)


def _dotm(a, x):
    return jax.lax.dot_general(a, x, (((1,), (0,)), ((), ())),
                               preferred_element_type=jnp.float32)


def _bf(x):
    return x.astype(jnp.bfloat16)


def _leaky(x):
    return jnp.where(x >= 0, x, LEAK * x)


def _block_rows(n, bm):
    return bm if n % bm == 0 else n


def _pre_body(su, ss, sv, tu, ts, tv, w1s, w2s, w3s, w1t, w2t, w3t,
              a_s, b_s, a_t, b_t):
    a_s[:, :F] = _bf(_dot(su[...], w1s[...]))
    a_s[:, F:] = _bf(_dot(ss[...], w2s[...]))
    b_s[...] = _bf(_dot(sv[...], w3s[...]))
    a_t[:, :F] = _bf(_dot(tu[...], w1t[...]))
    a_t[:, F:] = _bf(_dot(ts[...], w2t[...]))
    b_t[...] = _bf(_dot(tv[...], w3t[...]))


def _vu1_body(vu, a_in, bias, wg3m, wsg2, y_out):
    h = _leaky(_dotm(vu[...], a_in[...]) + bias[...])
    y_out[:, :F] = _bf(_dot(h[:, :F], wg3m[...]))
    y_out[:, F:] = _bf(_dot(h[:, F:], wsg2[...]))


def _uv_body(uv, b_in, y_in, bias_b, bias_y, ufea, wuum, buum, wsum, bsum,
             wg4m, user_out, d_out):
    blk = uv[...]
    item_ho = _leaky(_dotm(blk, b_in[...]) + bias_b[...])
    h = _leaky(_dotm(blk, y_in[...]) + bias_y[...])
    u_mean = h[:, :F]
    sh2 = h[:, F:]
    slu = _dot(u_mean, wuum[:F, :]) + _dot(ufea[...], wuum[F:, :]) + buum[...]
    csm = _dot(sh2, wsum[:F, :]) + _dot(slu, wsum[F:, :]) + bsum[...]
    user_out[...] = csm + slu
    d_out[...] = _bf(_dot(item_ho, wg4m[...]))


def _vu2_body(vu, d_in, bias, vfea, wium, bium, item_out):
    h = _leaky(_dotm(vu[...], d_in[...]) + bias[...])
    item_out[...] = _dot(h, wium[:F, :]) + _dot(vfea[...], wium[F:, :]) + bium[...]


def _const_spec(shape):
    return pl.BlockSpec(shape, lambda i: (0,) * len(shape))


def _row_spec(bm, width):
    return pl.BlockSpec((bm, width), lambda i: (i, 0))


def _row1(b):
    return b.reshape(1, -1)


def _domain(UV, VU, a_mat, b_mat, ufea, vfea, spec, cgc1, cgc2, cum):
    nu = ufea.shape[0]
    ni = vfea.shape[0]
    f32 = jnp.float32
    bf16 = jnp.bfloat16

    bm = _block_rows(ni, 400)
    bias1 = jnp.concatenate([_row1(spec["gc1"]["b"]), _row1(cgc1["b"])], axis=1)
    y_mat = pl.pallas_call(
        _vu1_body,
        grid=(ni // bm,),
        in_specs=[_row_spec(bm, nu), _const_spec((nu, 2 * F)),
                  _const_spec((1, 2 * F)), _const_spec((F, F)),
                  _const_spec((F, F))],
        out_specs=_row_spec(bm, 2 * F),
        out_shape=jax.ShapeDtypeStruct((ni, 2 * F), bf16),
    )(VU, a_mat, bias1, spec["gc3m"]["W"], cgc2["W"])

    bm = _block_rows(nu, 400)
    bias_y = jnp.concatenate([_row1(spec["gc3m"]["b"]), _row1(cgc2["b"])], axis=1)
    user_out, d_mat = pl.pallas_call(
        _uv_body,
        grid=(nu // bm,),
        in_specs=[_row_spec(bm, ni), _const_spec((ni, F)),
                  _const_spec((ni, 2 * F)), _const_spec((1, F)),
                  _const_spec((1, 2 * F)), _row_spec(bm, F),
                  _const_spec((2 * F, F)), _const_spec((1, F)),
                  _const_spec((2 * F, F)), _const_spec((1, F)),
                  _const_spec((F, F))],
        out_specs=[_row_spec(bm, F), _row_spec(bm, F)],
        out_shape=[jax.ShapeDtypeStruct((nu, F), f32),
                   jax.ShapeDtypeStruct((nu, F), bf16)],
    )(UV, b_mat, y_mat, _row1(spec["gc2"]["b"]), bias_y, ufea,
      spec["uum"]["W"], _row1(spec["uum"]["b"]),
      cum["W"], _row1(cum["b"]), spec["gc4m"]["W"])

    bm = _block_rows(ni, 400)
    item_out = pl.pallas_call(
        _vu2_body,
        grid=(ni // bm,),
        in_specs=[_row_spec(bm, nu), _const_spec((nu, F)),
                  _const_spec((1, F)), _row_spec(bm, F),
                  _const_spec((2 * F, F)), _const_spec((1, F))],
        out_specs=_row_spec(bm, F),
        out_shape=jax.ShapeDtypeStruct((ni, F), f32),
    )(VU, d_mat, _row1(spec["gc4m"]["b"]), vfea,
      spec["ium"]["W"], _row1(spec["ium"]["b"]))

    return user_out, item_out


@functools.partial(jax.jit)
def kernel(source_UV, source_VU, target_UV, target_VU, params):
    cond = params["cond"]
    bf16 = jnp.bfloat16
    s_spec, t_spec = params["src_specific"], params["tgt_specific"]
    su, sv = params["src_user_emb"], params["src_item_emb"]
    tu, tv = params["tgt_user_emb"], params["tgt_item_emb"]

    n = su.shape[0]
    bm = _block_rows(n, 1000)
    a_s, b_s, a_t, b_t = pl.pallas_call(
        _pre_body,
        grid=(n // bm,),
        in_specs=[_row_spec(bm, F)] * 6 + [_const_spec((F, F))] * 6,
        out_specs=[_row_spec(bm, 2 * F), _row_spec(bm, F),
                   _row_spec(bm, 2 * F), _row_spec(bm, F)],
        out_shape=[jax.ShapeDtypeStruct((su.shape[0], 2 * F), bf16),
                   jax.ShapeDtypeStruct((sv.shape[0], F), bf16),
                   jax.ShapeDtypeStruct((tu.shape[0], 2 * F), bf16),
                   jax.ShapeDtypeStruct((tv.shape[0], F), bf16)],
    )(su, params["src_user_share"], sv, tu, params["tgt_user_share"], tv,
      s_spec["gc1"]["W"], cond["s_gc1"]["W"], s_spec["gc2"]["W"],
      t_spec["gc1"]["W"], cond["t_gc1"]["W"], t_spec["gc2"]["W"])

    s_user, s_item = _domain(source_UV, source_VU, a_s, b_s, su, sv,
                             s_spec, cond["s_gc1"], cond["s_gc2"], cond["s_um"])
    t_user, t_item = _domain(target_UV, target_VU, a_t, b_t, tu, tv,
                             t_spec, cond["t_gc1"], cond["t_gc2"], cond["t_um"])
    return s_user, s_item, t_user, t_item


# speedup vs baseline: 1.0729x; 1.0094x over previous
import functools

import jax
import jax.numpy as jnp
from jax.experimental import pallas as pl
from jax.experimental.pallas import tpu as pltpu

F = 128
LEAK = 0.1


def _dot(a, b):
    return jnp.dot(a, b, preferred_element_type=jnp.float32)


def _dotm(a, x):
    return jax.lax.dot_general(a, x, (((1,), (0,)), ((), ())),
                               preferred_element_type=jnp.float32)


def _bf(x):
    return x.astype(jnp.bfloat16)


def _leaky(x):
    return jnp.where(x >= 0, x, LEAK * x)


def _block_rows(n, bm):
    return bm if n % bm == 0 else n


def _vu1_body(vu, ufea, share, vfea, w1, w2, wg2, bias, wg3m, wsg2,
              y_out, b_out, a_scr):
    i = pl.program_id(0)

    @pl.when(i == 0)
    def _():
        a_scr[:, :F] = _bf(_dot(ufea[...], w1[...]))
        a_scr[:, F:] = _bf(_dot(share[...], w2[...]))

    @pl.when(i > 0)
    def _():
        h = _leaky(_dotm(vu[...], a_scr[...]) + bias[...])
        y_out[:, :F] = _bf(_dot(h[:, :F], wg3m[...]))
        y_out[:, F:] = _bf(_dot(h[:, F:], wsg2[...]))
        b_out[...] = _bf(_dot(vfea[...], wg2[...]))


def _uv_body(uv, b_in, y_in, bias_b, bias_y, ufea, wuum, buum, wsum, bsum,
             wg4m, user_out, d_out):
    blk = uv[...]
    item_ho = _leaky(_dotm(blk, b_in[...]) + bias_b[...])
    h = _leaky(_dotm(blk, y_in[...]) + bias_y[...])
    u_mean = h[:, :F]
    sh2 = h[:, F:]
    slu = _dot(u_mean, wuum[:F, :]) + _dot(ufea[...], wuum[F:, :]) + buum[...]
    csm = _dot(sh2, wsum[:F, :]) + _dot(slu, wsum[F:, :]) + bsum[...]
    user_out[...] = csm + slu
    d_out[...] = _bf(_dot(item_ho, wg4m[...]))


def _vu2_body(vu, d_in, bias, vfea, wium, bium, item_out):
    h = _leaky(_dotm(vu[...], d_in[...]) + bias[...])
    item_out[...] = _dot(h, wium[:F, :]) + _dot(vfea[...], wium[F:, :]) + bium[...]


def _const_spec(shape):
    return pl.BlockSpec(shape, lambda i: (0,) * len(shape))


def _row_spec(bm, width):
    return pl.BlockSpec((bm, width), lambda i: (i, 0))


def _row1(b):
    return b.reshape(1, -1)


def _domain(UV, VU, share, ufea, vfea, spec, cgc1, cgc2, cum):
    nu = ufea.shape[0]
    ni = vfea.shape[0]
    f32 = jnp.float32
    bf16 = jnp.bfloat16

    bm = _block_rows(ni, 400)
    bias1 = jnp.concatenate([_row1(spec["gc1"]["b"]), _row1(cgc1["b"])], axis=1)

    def _lag(bm_, width):
        return pl.BlockSpec((bm_, width),
                            lambda i: (jnp.maximum(i - 1, 0), 0))

    y_mat, b_mat = pl.pallas_call(
        _vu1_body,
        grid=(ni // bm + 1,),
        in_specs=[_lag(bm, nu), _const_spec((nu, F)), _const_spec((nu, F)),
                  _lag(bm, F), _const_spec((F, F)), _const_spec((F, F)),
                  _const_spec((F, F)), _const_spec((1, 2 * F)),
                  _const_spec((F, F)), _const_spec((F, F))],
        out_specs=[_lag(bm, 2 * F), _lag(bm, F)],
        out_shape=[jax.ShapeDtypeStruct((ni, 2 * F), bf16),
                   jax.ShapeDtypeStruct((ni, F), bf16)],
        scratch_shapes=[pltpu.VMEM((nu, 2 * F), bf16)],
    )(VU, ufea, share, vfea, spec["gc1"]["W"], cgc1["W"], spec["gc2"]["W"],
      bias1, spec["gc3m"]["W"], cgc2["W"])

    bm = _block_rows(nu, 400)
    bias_y = jnp.concatenate([_row1(spec["gc3m"]["b"]), _row1(cgc2["b"])], axis=1)
    user_out, d_mat = pl.pallas_call(
        _uv_body,
        grid=(nu // bm,),
        in_specs=[_row_spec(bm, ni), _const_spec((ni, F)),
                  _const_spec((ni, 2 * F)), _const_spec((1, F)),
                  _const_spec((1, 2 * F)), _row_spec(bm, F),
                  _const_spec((2 * F, F)), _const_spec((1, F)),
                  _const_spec((2 * F, F)), _const_spec((1, F)),
                  _const_spec((F, F))],
        out_specs=[_row_spec(bm, F), _row_spec(bm, F)],
        out_shape=[jax.ShapeDtypeStruct((nu, F), f32),
                   jax.ShapeDtypeStruct((nu, F), bf16)],
    )(UV, b_mat, y_mat, _row1(spec["gc2"]["b"]), bias_y, ufea,
      spec["uum"]["W"], _row1(spec["uum"]["b"]),
      cum["W"], _row1(cum["b"]), spec["gc4m"]["W"])

    bm = _block_rows(ni, 400)
    item_out = pl.pallas_call(
        _vu2_body,
        grid=(ni // bm,),
        in_specs=[_row_spec(bm, nu), _const_spec((nu, F)),
                  _const_spec((1, F)), _row_spec(bm, F),
                  _const_spec((2 * F, F)), _const_spec((1, F))],
        out_specs=_row_spec(bm, F),
        out_shape=jax.ShapeDtypeStruct((ni, F), f32),
    )(VU, d_mat, _row1(spec["gc4m"]["b"]), vfea,
      spec["ium"]["W"], _row1(spec["ium"]["b"]))

    return user_out, item_out


@functools.partial(jax.jit)
def kernel(source_UV, source_VU, target_UV, target_VU, params):
    cond = params["cond"]
    s_spec, t_spec = params["src_specific"], params["tgt_specific"]

    s_user, s_item = _domain(source_UV, source_VU, params["src_user_share"],
                             params["src_user_emb"], params["src_item_emb"],
                             s_spec, cond["s_gc1"], cond["s_gc2"], cond["s_um"])
    t_user, t_item = _domain(target_UV, target_VU, params["tgt_user_share"],
                             params["tgt_user_emb"], params["tgt_item_emb"],
                             t_spec, cond["t_gc1"], cond["t_gc2"], cond["t_um"])
    return s_user, s_item, t_user, t_item


# speedup vs baseline: 1.1027x; 1.0278x over previous
import functools

import jax
import jax.numpy as jnp
from jax.experimental import pallas as pl
from jax.experimental.pallas import tpu as pltpu

F = 128
LEAK = 0.1


def _dot(a, b):
    return jnp.dot(a, b, preferred_element_type=jnp.float32)


def _dotm(a, x):
    return jax.lax.dot_general(a, x, (((1,), (0,)), ((), ())),
                               preferred_element_type=jnp.float32)


def _bf(x):
    return x.astype(jnp.bfloat16)


def _leaky(x):
    return jnp.where(x >= 0, x, LEAK * x)


def _row1(b):
    return b.reshape(1, -1)


def _domain_body(nblk, bm,
                 vu, uv, ufea_f, share_f, vfea_b, ufea_b,
                 wg1, wsg1, wg2, wg3m, wsg2, wuum, wsum, wg4m, wium,
                 bias1, bias_b, bias_y, bias4, buum, bsum, bium,
                 user_out, item_out,
                 abuf, a_scr, y_scr, b_scr, d_scr, sem):
    i = pl.program_id(0)

    @pl.when(i < 3 * nblk)
    def _():
        pblk = i % nblk
        slot = i % 2

        @pl.when(i // nblk != 1)
        def _():
            pltpu.make_async_copy(vu.at[pl.ds(pblk * bm, bm), :],
                                  abuf.at[slot], sem.at[slot]).start()

        @pl.when(i // nblk == 1)
        def _():
            pltpu.make_async_copy(uv.at[pl.ds(pblk * bm, bm), :],
                                  abuf.at[slot], sem.at[slot]).start()

    @pl.when(i == 0)
    def _():
        a_scr[:, :F] = _bf(_dot(ufea_f[...], wg1[...]))
        a_scr[:, F:] = _bf(_dot(share_f[...], wsg1[...]))

    def _phases(adj_ref, p, blk):
        @pl.when(p == 0)
        def _():
            h = _leaky(_dotm(adj_ref[...], a_scr[...]) + bias1[...])
            y_scr[pl.ds(blk * bm, bm), :F] = _bf(_dot(h[:, :F], wg3m[...]))
            y_scr[pl.ds(blk * bm, bm), F:] = _bf(_dot(h[:, F:], wsg2[...]))
            b_scr[pl.ds(blk * bm, bm), :] = _bf(_dot(vfea_b[...], wg2[...]))

        @pl.when(p == 1)
        def _():
            item_ho = _leaky(_dotm(adj_ref[...], b_scr[...]) + bias_b[...])
            h = _leaky(_dotm(adj_ref[...], y_scr[...]) + bias_y[...])
            u_mean = h[:, :F]
            sh2 = h[:, F:]
            slu = (_dot(u_mean, wuum[:F, :]) + _dot(ufea_b[...], wuum[F:, :])
                   + buum[...])
            csm = (_dot(sh2, wsum[:F, :]) + _dot(slu, wsum[F:, :])
                   + bsum[...])
            user_out[...] = csm + slu
            d_scr[pl.ds(blk * bm, bm), :] = _bf(_dot(item_ho, wg4m[...]))

        @pl.when(p == 2)
        def _():
            hh = _leaky(_dotm(adj_ref[...], d_scr[...]) + bias4[...])
            item_out[...] = (_dot(hh, wium[:F, :]) + _dot(vfea_b[...],
                                                          wium[F:, :])
                             + bium[...])

    @pl.when(i > 0)
    def _():
        j = i - 1
        p = j // nblk
        blk = j % nblk
        slot = j % 2
        pltpu.make_async_copy(vu.at[pl.ds(blk * bm, bm), :],
                              abuf.at[slot], sem.at[slot]).wait()

        @pl.when(slot == 0)
        def _():
            _phases(abuf.at[0], p, blk)

        @pl.when(slot == 1)
        def _():
            _phases(abuf.at[1], p, blk)


def _domain(UV, VU, share, ufea, vfea, spec, cgc1, cgc2, cum):
    nu = ufea.shape[0]
    ni = vfea.shape[0]
    f32 = jnp.float32
    bf16 = jnp.bfloat16
    bm = 400 if ni % 400 == 0 else ni
    nblk = ni // bm

    def _const(shape):
        return pl.BlockSpec(shape, lambda i: (0,) * len(shape))

    vfea_spec = pl.BlockSpec(
        (bm, F),
        lambda i: (jnp.clip(jnp.where(i - 1 >= 2 * nblk, i - 1 - 2 * nblk,
                                      i - 1), 0, nblk - 1), 0))
    ufea_spec = pl.BlockSpec(
        (bm, F), lambda i: (jnp.clip(i - 1 - nblk, 0, nblk - 1), 0))

    bias1 = jnp.concatenate([_row1(spec["gc1"]["b"]), _row1(cgc1["b"])],
                            axis=1)
    bias_y = jnp.concatenate([_row1(spec["gc3m"]["b"]), _row1(cgc2["b"])],
                             axis=1)

    user_out, item_out = pl.pallas_call(
        functools.partial(_domain_body, nblk, bm),
        grid=(3 * nblk + 1,),
        in_specs=[pl.BlockSpec(memory_space=pl.ANY),
                  pl.BlockSpec(memory_space=pl.ANY),
                  _const((nu, F)), _const((nu, F)),
                  vfea_spec, ufea_spec,
                  _const((F, F)), _const((F, F)), _const((F, F)),
                  _const((F, F)), _const((F, F)),
                  _const((2 * F, F)), _const((2 * F, F)), _const((F, F)),
                  _const((2 * F, F)),
                  _const((1, 2 * F)), _const((1, F)), _const((1, 2 * F)),
                  _const((1, F)), _const((1, F)), _const((1, F)),
                  _const((1, F))],
        out_specs=[
            pl.BlockSpec((bm, F),
                         lambda i: (jnp.clip(i - 1 - nblk, 0, nblk - 1), 0)),
            pl.BlockSpec((bm, F),
                         lambda i: (jnp.clip(i - 1 - 2 * nblk, 0, nblk - 1),
                                    0))],
        out_shape=[jax.ShapeDtypeStruct((nu, F), f32),
                   jax.ShapeDtypeStruct((ni, F), f32)],
        scratch_shapes=[pltpu.VMEM((2, bm, nu), f32),
                        pltpu.VMEM((nu, 2 * F), bf16),
                        pltpu.VMEM((ni, 2 * F), bf16),
                        pltpu.VMEM((ni, F), bf16),
                        pltpu.VMEM((nu, F), bf16),
                        pltpu.SemaphoreType.DMA((2,))],
        compiler_params=pltpu.CompilerParams(
            vmem_limit_bytes=63 * 1024 * 1024),
    )(VU, UV, ufea, share, vfea, ufea,
      spec["gc1"]["W"], cgc1["W"], spec["gc2"]["W"], spec["gc3m"]["W"],
      cgc2["W"], spec["uum"]["W"], cum["W"], spec["gc4m"]["W"],
      spec["ium"]["W"],
      bias1, _row1(spec["gc2"]["b"]), bias_y, _row1(spec["gc4m"]["b"]),
      _row1(spec["uum"]["b"]), _row1(cum["b"]), _row1(spec["ium"]["b"]))

    return user_out, item_out


@functools.partial(jax.jit)
def kernel(source_UV, source_VU, target_UV, target_VU, params):
    cond = params["cond"]
    s_spec, t_spec = params["src_specific"], params["tgt_specific"]

    s_user, s_item = _domain(source_UV, source_VU, params["src_user_share"],
                             params["src_user_emb"], params["src_item_emb"],
                             s_spec, cond["s_gc1"], cond["s_gc2"],
                             cond["s_um"])
    t_user, t_item = _domain(target_UV, target_VU, params["tgt_user_share"],
                             params["tgt_user_emb"], params["tgt_item_emb"],
                             t_spec, cond["t_gc1"], cond["t_gc2"],
                             cond["t_um"])
    return s_user, s_item, t_user, t_item
